# Initial kernel scaffold; baseline (speedup 1.0000x reference)
import jax
import jax.numpy as jnp
from jax.experimental import pallas as pl


def kernel(x, edge_index, num_nodes_cum, W_in, b_in, W_z, W_u, es_W1, es_b1, es_W2, es_b2, es_W3, es_b3, sm_W1, sm_b1, sm_W2, sm_b2, sm_W3, sm_b3, em_W1, em_b1, em_W2, em_b2, em_W3, em_b3, sf_W1, sf_b1, sf_W2, sf_b2, sf_W3, sf_b3, W_ef, W_fm):
    raise NotImplementedError("write your pallas kernel here")


# speedup vs baseline: 6.0215x; 6.0215x over previous
import jax
import jax.numpy as jnp
from jax.experimental import pallas as pl

_BN = 1000
_BE = 1000


def _rep(shape):
    return pl.BlockSpec(shape, lambda i: (0,) * len(shape))


def _blk(b, d):
    return pl.BlockSpec((b, d), lambda i: (i, 0))


def _prep_kernel(x_ref, win_ref, bin_ref, wz_ref, s_ref, f_ref, z_ref):
    xb = x_ref[...]
    h = jnp.concatenate([xb[:, :8], xb[:, 26:]], axis=1)
    s_ref[...] = h @ win_ref[...] + bin_ref[...]
    cols = []
    for a in range(3):
        for m in range(6):
            cols.append(xb[:, 8 + 3 * m + a:8 + 3 * m + a + 1])
    z = jnp.concatenate(cols, axis=1)
    z_ref[...] = z
    wz = wz_ref[...]
    f_ref[...] = jnp.concatenate(
        [z[:, 6 * a:6 * a + 6] @ wz for a in range(3)], axis=1)


def _edge_kernel(se0_ref, se1_ref, fe0_ref, fe1_ref, eaf_ref,
                 w1a_ref, w1b_ref, w1c_ref, b1_ref, w2_ref, b2_ref,
                 w3_ref, b3_ref,
                 ew1_ref, eb1_ref, ew2_ref, eb2_ref, ew3_ref, eb3_ref,
                 wef_ref, s_out_ref, fe_out_ref):
    eaf = eaf_ref[...]
    eas = jnp.sqrt(eaf[:, 0:1] ** 2 + eaf[:, 6:7] ** 2 + eaf[:, 12:13] ** 2)
    h = jax.nn.silu(se0_ref[...] @ w1a_ref[...] + se1_ref[...] @ w1b_ref[...]
                    + eas * w1c_ref[...] + b1_ref[...])
    h = jax.nn.silu(h @ w2_ref[...] + b2_ref[...])
    _s = h @ w3_ref[...] + b3_ref[...]
    s_out_ref[...] = _s
    g = jax.nn.silu(_s @ ew1_ref[...] + eb1_ref[...])
    g = jax.nn.silu(g @ ew2_ref[...] + eb2_ref[...])
    em = g @ ew3_ref[...] + eb3_ref[...]
    wef = wef_ref[...]
    fe0 = fe0_ref[...]
    fe1 = fe1_ref[...]
    outs = []
    for a in range(3):
        t = (fe0[:, 16 * a:16 * a + 16] @ wef[0:16]
             + fe1[:, 16 * a:16 * a + 16] @ wef[16:32]
             + eaf[:, 6 * a:6 * a + 6] @ wef[32:38])
        outs.append(em * t)
    fe_out_ref[...] = jnp.concatenate(outs, axis=1)


def _node_kernel(s_ref, ssum_ref, f_ref, fsum_ref, cnt_ref,
                 mw1a_ref, mw1b_ref, mw1c_ref, mb1_ref, mw2_ref, mb2_ref,
                 mw3_ref, mb3_ref,
                 fw1a_ref, fw1b_ref, fw1c_ref, fb1_ref, fw2_ref, fb2_ref,
                 fw3_ref, fb3_ref,
                 wfm_ref, s_out_ref, f_out_ref):
    inv = 1.0 / jnp.maximum(cnt_ref[...], 1.0)
    s_c = ssum_ref[...] * inv
    f_c = fsum_ref[...] * inv
    fv = f_ref[...]
    blocks = []
    for p in range(16):
        acc = fv[:, p:p + 1] * fv[:, 0:16]
        acc = acc + fv[:, 16 + p:17 + p] * fv[:, 16:32]
        acc = acc + fv[:, 32 + p:33 + p] * fv[:, 32:48]
        blocks.append(acc)
    f2s = jnp.concatenate(blocks, axis=1)
    sv = s_ref[...]
    h = jax.nn.silu(sv @ mw1a_ref[...] + s_c @ mw1b_ref[...]
                    + f2s @ mw1c_ref[...] + mb1_ref[...])
    h = jax.nn.silu(h @ mw2_ref[...] + mb2_ref[...])
    s_out_ref[...] = h @ mw3_ref[...] + mb3_ref[...]
    g = jax.nn.silu(sv @ fw1a_ref[...] + s_c @ fw1b_ref[...]
                    + f2s @ fw1c_ref[...] + fb1_ref[...])
    g = jax.nn.silu(g @ fw2_ref[...] + fb2_ref[...])
    sf = g @ fw3_ref[...] + fb3_ref[...]
    wfm = wfm_ref[...]
    outs = []
    for a in range(3):
        t = (fv[:, 16 * a:16 * a + 16] @ wfm[0:16]
             + f_c[:, 16 * a:16 * a + 16] @ wfm[16:32])
        outs.append(sf * t)
    f_out_ref[...] = jnp.concatenate(outs, axis=1)


def _ab_kernel(f_ref, wu_ref, ab_ref):
    fv = f_ref[...]
    wu = wu_ref[...]
    u0 = fv[:, 0:16] @ wu
    u1 = fv[:, 16:32] @ wu
    nrm = jnp.sqrt(u0 * u0 + u1 * u1) + 1e-6
    ab_ref[...] = jnp.concatenate([u0 / nrm, u1 / nrm], axis=1)


def _rot_kernel(ab_ref, z_ref, out_ref):
    ab = ab_ref[...]
    a = ab[:, 0:1]
    b = ab[:, 1:2]
    z = z_ref[...]
    r0 = a * z[:, 0:6] + b * z[:, 6:12]
    r1 = -b * z[:, 0:6] + a * z[:, 6:12]
    r2 = z[:, 12:18]
    cols = []
    for k in range(6):
        cols += [r0[:, k:k + 1], r1[:, k:k + 1], r2[:, k:k + 1]]
    out_ref[...] = jnp.concatenate(cols, axis=1)


def kernel(x, edge_index, num_nodes_cum, W_in, b_in, W_z, W_u,
           es_W1, es_b1, es_W2, es_b2, es_W3, es_b3,
           sm_W1, sm_b1, sm_W2, sm_b2, sm_W3, sm_b3,
           em_W1, em_b1, em_W2, em_b2, em_W3, em_b3,
           sf_W1, sf_b1, sf_W2, sf_b2, sf_W3, sf_b3,
           W_ef, W_fm):
    n = x.shape[0]
    e0 = edge_index[0]
    e1 = edge_index[1]
    ne = e0.shape[0]
    f32 = jnp.float32

    def r2(v):
        return v.reshape(1, -1).astype(f32)

    grid_n = pl.cdiv(n, _BN)
    grid_e = pl.cdiv(ne, _BE)

    s, f, z = pl.pallas_call(
        _prep_kernel,
        grid=(grid_n,),
        in_specs=[_blk(_BN, 48), _rep((30, 64)), _rep((1, 64)), _rep((6, 16))],
        out_specs=[_blk(_BN, 64), _blk(_BN, 48), _blk(_BN, 18)],
        out_shape=[jax.ShapeDtypeStruct((n, 64), f32),
                   jax.ShapeDtypeStruct((n, 48), f32),
                   jax.ShapeDtypeStruct((n, 18), f32)],
    )(x, W_in, r2(b_in), W_z)

    h0 = jnp.concatenate([x[:, :8], x[:, 26:]], axis=1)
    eaf = jnp.take(z, e1, axis=0) - jnp.take(z, e0, axis=0)
    cnt = jax.ops.segment_sum(jnp.ones((ne, 1), f32), e0, num_segments=n)

    edge_w = (es_W1[:64], es_W1[64:128], es_W1[128:129], r2(es_b1),
              es_W2, r2(es_b2), es_W3, r2(es_b3),
              em_W1, r2(em_b1), em_W2, r2(em_b2), em_W3, r2(em_b3), W_ef)
    node_w = (sm_W1[:64], sm_W1[64:128], sm_W1[128:], r2(sm_b1),
              sm_W2, r2(sm_b2), sm_W3, r2(sm_b3),
              sf_W1[:64], sf_W1[64:128], sf_W1[128:], r2(sf_b1),
              sf_W2, r2(sf_b2), sf_W3, r2(sf_b3), W_fm)

    edge_call = pl.pallas_call(
        _edge_kernel,
        grid=(grid_e,),
        in_specs=[_blk(_BE, 64), _blk(_BE, 64), _blk(_BE, 48), _blk(_BE, 48),
                  _blk(_BE, 18)] + [_rep(w.shape) for w in edge_w],
        out_specs=[_blk(_BE, 64), _blk(_BE, 48)],
        out_shape=[jax.ShapeDtypeStruct((ne, 64), f32),
                   jax.ShapeDtypeStruct((ne, 48), f32)],
    )
    node_call = pl.pallas_call(
        _node_kernel,
        grid=(grid_n,),
        in_specs=[_blk(_BN, 64), _blk(_BN, 64), _blk(_BN, 48), _blk(_BN, 48),
                  _blk(_BN, 1)] + [_rep(w.shape) for w in node_w],
        out_specs=[_blk(_BN, 64), _blk(_BN, 48)],
        out_shape=[jax.ShapeDtypeStruct((n, 64), f32),
                   jax.ShapeDtypeStruct((n, 48), f32)],
    )

    for _ in range(2):
        se0 = jnp.take(s, e0, axis=0)
        se1 = jnp.take(s, e1, axis=0)
        fe0 = jnp.take(f, e0, axis=0)
        fe1 = jnp.take(f, e1, axis=0)
        _s, _fe = edge_call(se0, se1, fe0, fe1, eaf, *edge_w)
        s_sum = jax.ops.segment_sum(_s, e0, num_segments=n)
        f_sum = jax.ops.segment_sum(_fe, e0, num_segments=n)
        s, f = node_call(s, s_sum, f, f_sum, cnt, *node_w)

    ab = pl.pallas_call(
        _ab_kernel,
        grid=(grid_n,),
        in_specs=[_blk(_BN, 48), _rep((16, 1))],
        out_specs=_blk(_BN, 2),
        out_shape=jax.ShapeDtypeStruct((n, 2), f32),
    )(f, W_u)

    starts = jnp.concatenate(
        [jnp.zeros((1,), num_nodes_cum.dtype), num_nodes_cum[:-1]])
    counts = num_nodes_cum - starts
    root = jnp.repeat(starts, counts, total_repeat_length=n)
    ab_r = jnp.take(ab, root, axis=0)

    fp2 = pl.pallas_call(
        _rot_kernel,
        grid=(grid_n,),
        in_specs=[_blk(_BN, 2), _blk(_BN, 18)],
        out_specs=_blk(_BN, 18),
        out_shape=jax.ShapeDtypeStruct((n, 18), f32),
    )(ab_r, z)

    return jnp.concatenate([fp2, h0], axis=1)
